# baseline (speedup 1.0000x reference)
import functools

import jax
import jax.numpy as jnp
from jax import lax
from jax.experimental import pallas as pl
from jax.experimental.pallas import tpu as pltpu
from jax.experimental.pallas import tpu_sc as plsc

B = 64
N = 128
E = 2048
G = 8


def _split(x):
    hi = x.astype(jnp.bfloat16)
    lo = (x - hi.astype(jnp.float32)).astype(jnp.bfloat16)
    return hi, lo


def _dot_oh(oh_bf16, x):
    hi, lo = _split(x)
    return (jnp.dot(oh_bf16, hi, preferred_element_type=jnp.float32)
            + jnp.dot(oh_bf16, lo, preferred_element_type=jnp.float32))


def _dot3(x, w):
    xh, xl = _split(x)
    wh, wl = _split(w)
    return (jnp.dot(xh, wh, preferred_element_type=jnp.float32)
            + jnp.dot(xh, wl, preferred_element_type=jnp.float32)
            + jnp.dot(xl, wh, preferred_element_type=jnp.float32))


def _dot3_pre(x, wh, wl):
    xh, xl = _split(x)
    return (jnp.dot(xh, wh, preferred_element_type=jnp.float32)
            + jnp.dot(xh, wl, preferred_element_type=jnp.float32)
            + jnp.dot(xl, wh, preferred_element_type=jnp.float32))


def _graph_body(eit_ref, ei_ref, x_ref,
                W1a_ref, b1a_ref, W1b_ref, b1b_ref,
                W2a_ref, b2a_ref, W2b_ref, b2b_ref,
                W3a_ref, b3a_ref, W3b_ref, b3b_ref,
                g1_ref, be1_ref, g2_ref, be2_ref, g3_ref, be3_ref,
                out_ref, av_s, bv_s):
    f32 = jnp.float32
    src_col = eit_ref[:, 0:1]
    dst_col = eit_ref[:, 1:2]
    dst_row = ei_ref[1:2, :]
    bf16 = jnp.bfloat16
    niota_r = lax.broadcasted_iota(jnp.int32, (E, N), 1)
    DS = jnp.concatenate([(dst_col == niota_r).astype(bf16),
                          (src_col == niota_r).astype(bf16)], axis=1)
    niota_c = lax.broadcasted_iota(jnp.int32, (N, E), 0)
    dmask = (niota_c == dst_row)
    DT = dmask.astype(bf16)
    deg = jnp.sum(dmask.astype(f32), axis=1, keepdims=True)

    def layer(h, fin, Wa_ref, ba_ref, Wb_ref, bb_ref, g_ref, be_ref):
        Wa = Wa_ref[...]
        Wd = Wa[:fin] - Wa[fin:]
        Wbot = Wa[fin:]
        ba = ba_ref[...]
        hf = h.reshape(B * N, fin)
        if fin == 2:
            av_s[...] = (jnp.dot(hf, Wd, preferred_element_type=f32, precision=lax.Precision.HIGHEST) + ba).reshape(B, N, 64)
            bv_s[...] = jnp.dot(hf, Wbot, preferred_element_type=f32, precision=lax.Precision.HIGHEST).reshape(B, N, 64)
        else:
            av_s[...] = (_dot3(hf, Wd) + ba).reshape(B, N, 64)
            bv_s[...] = _dot3(hf, Wbot).reshape(B, N, 64)

        Wb = Wb_ref[...]
        Z = jnp.zeros((64, 128), f32)
        WbG = jnp.concatenate(
            [jnp.concatenate([Wb if j == i else Z for j in range(G)], axis=1)
             for i in range(G)], axis=0)
        WbGh, WbGl = _split(WbG)
        bb = bb_ref[...]
        bbG = jnp.concatenate([bb] * G, axis=1)

        def grp(i, carry):
            a = jnp.concatenate(
                [av_s[pl.ds(i * G + j, 1), :, :].reshape(N, 64)
                 for j in range(G)], axis=1)
            bm = jnp.concatenate(
                [bv_s[pl.ds(i * G + j, 1), :, :].reshape(N, 64)
                 for j in range(G)], axis=1)
            pre = _dot_oh(DS, jnp.concatenate([a, bm], axis=0))
            r = jnp.maximum(pre, 0.0)
            agg = _dot_oh(DT, r)
            o = _dot3_pre(agg, WbGh, WbGl) + deg * bbG
            for j in range(G):
                out_ref[pl.ds(i * G + j, 1), :, :] = o[:, j * 128:(j + 1) * 128].reshape(1, N, 128)
            return carry

        lax.fori_loop(0, B // G, grp, 0)
        out = out_ref[...]
        inv = 1.0 / (B * 128)
        mu = jnp.sum(jnp.sum(out, axis=0, keepdims=True), axis=2, keepdims=True) * inv
        d = out - mu
        var = jnp.sum(jnp.sum(d * d, axis=0, keepdims=True), axis=2, keepdims=True) * inv
        hn = d * lax.rsqrt(var + 1e-5) * g_ref[...] + be_ref[...]
        return jnp.maximum(hn, 0.0)

    h = layer(x_ref[...], 2, W1a_ref, b1a_ref, W1b_ref, b1b_ref, g1_ref, be1_ref)
    h = layer(h, 128, W2a_ref, b2a_ref, W2b_ref, b2b_ref, g2_ref, be2_ref)
    h = layer(h, 128, W3a_ref, b3a_ref, W3b_ref, b3b_ref, g3_ref, be3_ref)
    out_ref[...] = h


def _head_body(hf_ref, Wm1_ref, bm1_ref, Wm2_ref, bm2_ref, Wm3_ref, bm3_ref,
               Wv_ref, bv_ref, Wadv_ref, badv_ref, out_ref, acc_ref):
    f32 = jnp.float32
    k = pl.program_id(0)

    @pl.when(k == 0)
    def _():
        acc_ref[...] = jnp.zeros_like(acc_ref)

    acc_ref[...] += _dot3(hf_ref[...], Wm1_ref[...])

    @pl.when(k == pl.num_programs(0) - 1)
    def _():
        z = jnp.maximum(acc_ref[...] + bm1_ref[...], 0.0)
        z = jnp.maximum(jnp.dot(z, Wm2_ref[...], preferred_element_type=f32, precision=lax.Precision.HIGHEST) + bm2_ref[...], 0.0)
        z = jnp.maximum(jnp.dot(z, Wm3_ref[...], preferred_element_type=f32, precision=lax.Precision.HIGHEST) + bm3_ref[...], 0.0)
        value = jnp.dot(z, Wv_ref[...], preferred_element_type=f32, precision=lax.Precision.HIGHEST) + bv_ref[...]
        adv = jnp.dot(z, Wadv_ref[...], preferred_element_type=f32, precision=lax.Precision.HIGHEST) + badv_ref[...]
        ii = lax.broadcasted_iota(jnp.int32, (64, 64), 0) // 2
        jj = lax.broadcasted_iota(jnp.int32, (64, 64), 1) // 2
        P = 0.5 * (ii == jj).astype(f32)
        out_ref[...] = value + adv - jnp.dot(adv, P, preferred_element_type=f32, precision=lax.Precision.HIGHEST)


SL = 16
EC = 16
NCH = (E // 16) // EC


def _sc_edge_body(av_hbm, bv_hbm, src_hbm, dst_hbm, z_hbm, out_hbm,
                  ga, gb, didx, sidx, dj, sj, At, Bt, acc, sem_a, sem_b):
    cid = lax.axis_index("c")
    sid = lax.axis_index("s")
    rows = pl.ds(sid * 8, 8)
    pltpu.sync_copy(av_hbm.at[cid, rows], At.at[rows])
    pltpu.sync_copy(bv_hbm.at[cid, rows], Bt.at[rows])
    pltpu.sync_copy(z_hbm, acc.at[rows])
    pltpu.sync_copy(dst_hbm.at[sid], didx)
    pltpu.sync_copy(src_hbm.at[sid], sidx)
    plsc.subcore_barrier()
    for j in range(NCH):
        dj[...] = didx[j]
        sj[...] = sidx[j]
        ca = pltpu.async_copy(At.at[dj], ga, sem_a)
        cb = pltpu.async_copy(Bt.at[sj], gb, sem_b)
        ca.wait()
        cb.wait()

        def body(s, carry):
            for r in range(EC):
                for l in range(8):
                    ga[r, s, pl.ds(l * 16, 16)] = jnp.maximum(
                        ga[r, s, pl.ds(l * 16, 16)]
                        + gb[r, s, pl.ds(l * 16, 16)], 0.0)
            return carry

        lax.fori_loop(0, SL, body, 0)
        pltpu.sync_copy(ga, acc.at[dj], add=True)
    plsc.subcore_barrier()
    pltpu.sync_copy(acc.at[rows], out_hbm.at[cid, rows])


def _sc_edge(av4, bv4, src3, dst3, zrows):
    f32 = jnp.float32
    mesh = plsc.VectorSubcoreMesh(core_axis_name="c", subcore_axis_name="s")
    k = pl.kernel(
        _sc_edge_body,
        out_type=jax.ShapeDtypeStruct((2, N, SL, 128), f32),
        mesh=mesh,
        scratch_types=[
            pltpu.VMEM((EC, SL, 128), f32),
            pltpu.VMEM((EC, SL, 128), f32),
            pltpu.VMEM((NCH, EC), jnp.int32),
            pltpu.VMEM((NCH, EC), jnp.int32),
            pltpu.VMEM((EC,), jnp.int32),
            pltpu.VMEM((EC,), jnp.int32),
            pltpu.VMEM_SHARED((N, SL, 128), f32),
            pltpu.VMEM_SHARED((N, SL, 128), f32),
            pltpu.VMEM_SHARED((N, SL, 128), f32),
            pltpu.SemaphoreType.DMA,
            pltpu.SemaphoreType.DMA,
        ],
    )
    return k(av4, bv4, src3, dst3, zrows)


def _tc_pre_body(x_ref, Wa_ref, ba_ref, av_o, bv_o):
    f32 = jnp.float32
    Wa = Wa_ref[...]
    xf = x_ref[...]
    av_o[...] = jnp.dot(xf, Wa[:2] - Wa[2:], preferred_element_type=f32,
                        precision=lax.Precision.HIGHEST) + ba_ref[...]
    bv_o[...] = jnp.dot(xf, Wa[2:], preferred_element_type=f32,
                        precision=lax.Precision.HIGHEST)


def _bn_from_partials(p_ref, ei_ref, Wb_ref, bb_ref, g_ref, be_ref):
    f32 = jnp.float32
    o = _dot3(p_ref[...], Wb_ref[...])
    dst_row = ei_ref[1:2, :]
    niota_c = lax.broadcasted_iota(jnp.int32, (N, E), 0)
    deg = jnp.sum((niota_c == dst_row).astype(f32), axis=1, keepdims=True)
    o = o.reshape(N, B, 128) + deg.reshape(N, 1, 1) * bb_ref[...]
    inv = 1.0 / (B * 128)
    mu = jnp.sum(jnp.sum(o, axis=1, keepdims=True), axis=2, keepdims=True) * inv
    d = o - mu
    var = jnp.sum(jnp.sum(d * d, axis=1, keepdims=True), axis=2, keepdims=True) * inv
    hn = d * lax.rsqrt(var + 1e-5) * g_ref[...] + be_ref[...]
    return jnp.maximum(hn, 0.0)


def _tc_mid_body(p_ref, ei_ref, Wb_ref, bb_ref, g_ref, be_ref,
                 Wa_ref, ba_ref, av_o, bv_o):
    h = _bn_from_partials(p_ref, ei_ref, Wb_ref, bb_ref, g_ref, be_ref)
    hf = h.reshape(N * B, 128)
    Wa = Wa_ref[...]
    av_o[...] = _dot3(hf, Wa[:128] - Wa[128:]) + ba_ref[...]
    bv_o[...] = _dot3(hf, Wa[128:])


def _tc_fin_body(p_ref, ei_ref, Wb_ref, bb_ref, g_ref, be_ref, h_o):
    h_o[...] = _bn_from_partials(p_ref, ei_ref, Wb_ref, bb_ref, g_ref, be_ref)


def _head_call(hf, Wm1, bm1, Wm2, bm2, Wm3, bm3, Wv, bv, Wadv, badv):
    f32 = jnp.float32
    Wadv_r = jnp.transpose(Wadv, (1, 0, 2)).reshape(256, 64)
    KC = 8
    KW = (N * N) // KC

    def cst(shape):
        return pl.BlockSpec(shape, lambda k: tuple(0 for _ in shape))

    return pl.pallas_call(
        _head_body,
        grid=(KC,),
        in_specs=[
            pl.BlockSpec((B, KW), lambda k: (0, k)),
            pl.BlockSpec((KW, 256), lambda k: (k, 0)),
            cst((1, 256)), cst((256, 256)), cst((1, 256)),
            cst((256, 256)), cst((1, 256)),
            cst((256, 1)), cst((1, 1)), cst((256, 64)), cst((1, 64)),
        ],
        out_specs=cst((B, 64)),
        out_shape=jax.ShapeDtypeStruct((B, 64), f32),
        scratch_shapes=[pltpu.VMEM((B, 256), f32)],
        compiler_params=pltpu.CompilerParams(
            dimension_semantics=("arbitrary",)),
    )(hf, Wm1, bm1.reshape(1, 256), Wm2, bm2.reshape(1, 256),
      Wm3, bm3.reshape(1, 256), Wv, bv.reshape(1, 1), Wadv_r, badv.reshape(1, 64))


def kernel(x, edge_index, W1a, b1a, W1b, b1b, W2a, b2a, W2b, b2b,
           W3a, b3a, W3b, b3b, g1, be1, g2, be2, g3, be3,
           Wm1, bm1, Wm2, bm2, Wm3, bm3, Wv, bv, Wadv, badv):
    f32 = jnp.float32
    x_nm = jnp.transpose(x, (1, 0, 2)).reshape(N * B, 2)
    src3 = edge_index[0].reshape(16, NCH, EC)
    dst3 = edge_index[1].reshape(16, NCH, EC)
    zrows = jnp.zeros((8, SL, 128), f32)

    def to_sc(t):
        return jnp.transpose(t.reshape(N, 2, SL, 128), (1, 0, 2, 3))

    def from_sc(p):
        return jnp.transpose(p, (1, 0, 2, 3)).reshape(N * B, 64)

    av, bvv = pl.pallas_call(
        _tc_pre_body,
        out_shape=(jax.ShapeDtypeStruct((N * B, 64), f32),
                   jax.ShapeDtypeStruct((N * B, 64), f32)),
    )(x_nm, W1a, b1a.reshape(1, 64))

    def mid(p, Wb, bb, g, be, Wa, ba):
        return pl.pallas_call(
            _tc_mid_body,
            out_shape=(jax.ShapeDtypeStruct((N * B, 64), f32),
                       jax.ShapeDtypeStruct((N * B, 64), f32)),
        )(from_sc(p), edge_index, Wb, bb.reshape(1, 1, 128),
          g.reshape(N, 1, 1), be.reshape(N, 1, 1), Wa, ba.reshape(1, 64))

    p = _sc_edge(to_sc(av), to_sc(bvv), src3, dst3, zrows)
    av, bvv = mid(p, W1b, b1b, g1, be1, W2a, b2a)
    p = _sc_edge(to_sc(av), to_sc(bvv), src3, dst3, zrows)
    av, bvv = mid(p, W2b, b2b, g2, be2, W3a, b3a)
    p = _sc_edge(to_sc(av), to_sc(bvv), src3, dst3, zrows)
    h3 = pl.pallas_call(
        _tc_fin_body,
        out_shape=jax.ShapeDtypeStruct((N, B, 128), f32),
    )(from_sc(p), edge_index, W3b, b3b.reshape(1, 1, 128),
      g3.reshape(N, 1, 1), be3.reshape(N, 1, 1))

    hf = jnp.transpose(h3, (1, 0, 2)).reshape(B, N * N)
    q = _head_call(hf, Wm1, bm1, Wm2, bm2, Wm3, bm3, Wv, bv, Wadv, badv)
    return q.reshape(B, 32, 2)


# speedup vs baseline: 1.4208x; 1.4208x over previous
import functools

import jax
import jax.numpy as jnp
from jax import lax
from jax.experimental import pallas as pl
from jax.experimental.pallas import tpu as pltpu
from jax.experimental.pallas import tpu_sc as plsc

B = 64
N = 128
E = 2048
G = 8


def _split(x):
    hi = x.astype(jnp.bfloat16)
    lo = (x - hi.astype(jnp.float32)).astype(jnp.bfloat16)
    return hi, lo


def _dot_oh(oh_bf16, x):
    hi, lo = _split(x)
    return (jnp.dot(oh_bf16, hi, preferred_element_type=jnp.float32)
            + jnp.dot(oh_bf16, lo, preferred_element_type=jnp.float32))


def _dot_oh3(oh_bf16, x):
    h1 = x.astype(jnp.bfloat16)
    r1 = x - h1.astype(jnp.float32)
    h2 = r1.astype(jnp.bfloat16)
    h3 = (r1 - h2.astype(jnp.float32)).astype(jnp.bfloat16)
    return (jnp.dot(oh_bf16, h1, preferred_element_type=jnp.float32)
            + jnp.dot(oh_bf16, h2, preferred_element_type=jnp.float32)
            + jnp.dot(oh_bf16, h3, preferred_element_type=jnp.float32))


def _dot3(x, w):
    xh, xl = _split(x)
    wh, wl = _split(w)
    return (jnp.dot(xh, wh, preferred_element_type=jnp.float32)
            + jnp.dot(xh, wl, preferred_element_type=jnp.float32)
            + jnp.dot(xl, wh, preferred_element_type=jnp.float32))


def _dot3_pre(x, wh, wl):
    xh, xl = _split(x)
    return (jnp.dot(xh, wh, preferred_element_type=jnp.float32)
            + jnp.dot(xh, wl, preferred_element_type=jnp.float32)
            + jnp.dot(xl, wh, preferred_element_type=jnp.float32))


S_SC = 32
B_TC = B - S_SC


def _tc_edge_body(eit_ref, ei_ref, av_ref, bv_ref, out_ref):
    bf16 = jnp.bfloat16
    src_col = eit_ref[:, 0:1]
    dst_col = eit_ref[:, 1:2]
    dst_row = ei_ref[1:2, :]
    niota_r = lax.broadcasted_iota(jnp.int32, (E, N), 1)
    DS = jnp.concatenate([(dst_col == niota_r).astype(bf16),
                          (src_col == niota_r).astype(bf16)], axis=1)
    niota_c = lax.broadcasted_iota(jnp.int32, (N, E), 0)
    DT = (niota_c == dst_row).astype(bf16)
    for g in range(B_TC // G):
        a = jnp.concatenate(
            [av_ref[g * G + j] for j in range(G)], axis=1)
        bm = jnp.concatenate(
            [bv_ref[g * G + j] for j in range(G)], axis=1)
        pre = _dot_oh3(DS, jnp.concatenate([a, bm], axis=0))
        r = jnp.maximum(pre, 0.0)
        agg = _dot_oh3(DT, r)
        out_ref[:, pl.ds(g * G * 64, G * 64)] = agg


def _tc_edge(eit, ei, av_tc, bv_tc):
    return pl.pallas_call(
        _tc_edge_body,
        out_shape=jax.ShapeDtypeStruct((N, B_TC * 64), jnp.float32),
    )(eit, ei, av_tc, bv_tc)


def _head_body(hf_ref, Wm1_ref, bm1_ref, Wm2_ref, bm2_ref, Wm3_ref, bm3_ref,
               Wv_ref, bv_ref, Wadv_ref, badv_ref, out_ref, acc_ref):
    f32 = jnp.float32
    k = pl.program_id(0)

    @pl.when(k == 0)
    def _():
        acc_ref[...] = jnp.zeros_like(acc_ref)

    acc_ref[...] += _dot3(hf_ref[...], Wm1_ref[...])

    @pl.when(k == pl.num_programs(0) - 1)
    def _():
        z = jnp.maximum(acc_ref[...] + bm1_ref[...], 0.0)
        z = jnp.maximum(jnp.dot(z, Wm2_ref[...], preferred_element_type=f32, precision=lax.Precision.HIGHEST) + bm2_ref[...], 0.0)
        z = jnp.maximum(jnp.dot(z, Wm3_ref[...], preferred_element_type=f32, precision=lax.Precision.HIGHEST) + bm3_ref[...], 0.0)
        value = jnp.dot(z, Wv_ref[...], preferred_element_type=f32, precision=lax.Precision.HIGHEST) + bv_ref[...]
        adv = jnp.dot(z, Wadv_ref[...], preferred_element_type=f32, precision=lax.Precision.HIGHEST) + badv_ref[...]
        ii = lax.broadcasted_iota(jnp.int32, (64, 64), 0) // 2
        jj = lax.broadcasted_iota(jnp.int32, (64, 64), 1) // 2
        P = 0.5 * (ii == jj).astype(f32)
        out_ref[...] = value + adv - jnp.dot(adv, P, preferred_element_type=f32, precision=lax.Precision.HIGHEST)


SL = S_SC // 4
EC = 16
NCH = (E // 16) // EC


def _sc_edge_body(av_hbm, bv_hbm, src_hbm, dst_hbm, z_hbm, out_hbm,
                  ga, gb, didx, sidx, dj, sj, At, Bt, acc, sem_a, sem_b):
    cid = lax.axis_index("c")
    sid = lax.axis_index("s")
    rows = pl.ds(sid * 8, 8)
    pltpu.sync_copy(av_hbm.at[cid, rows], At.at[rows])
    pltpu.sync_copy(bv_hbm.at[cid, rows], Bt.at[rows])
    pltpu.sync_copy(z_hbm, acc.at[rows])
    pltpu.sync_copy(dst_hbm.at[sid], didx)
    pltpu.sync_copy(src_hbm.at[sid], sidx)
    plsc.subcore_barrier()
    for j in range(NCH):
        dj[...] = didx[j]
        sj[...] = sidx[j]
        ca = pltpu.async_copy(At.at[dj], ga, sem_a)
        cb = pltpu.async_copy(Bt.at[sj], gb, sem_b)
        ca.wait()
        cb.wait()

        def body(s, carry):
            for r in range(EC):
                for l in range(8):
                    ga[r, s, pl.ds(l * 16, 16)] = jnp.maximum(
                        ga[r, s, pl.ds(l * 16, 16)]
                        + gb[r, s, pl.ds(l * 16, 16)], 0.0)
            return carry

        lax.fori_loop(0, SL, body, 0)
        pltpu.sync_copy(ga, acc.at[dj], add=True)
    plsc.subcore_barrier()
    pltpu.sync_copy(acc.at[rows], out_hbm.at[cid, rows])


def _sc_edge(av4, bv4, src3, dst3, zrows):
    f32 = jnp.float32
    mesh = plsc.VectorSubcoreMesh(core_axis_name="c", subcore_axis_name="s")
    k = pl.kernel(
        _sc_edge_body,
        out_type=jax.ShapeDtypeStruct((2, N, SL, 128), f32),
        mesh=mesh,
        scratch_types=[
            pltpu.VMEM((EC, SL, 128), f32),
            pltpu.VMEM((EC, SL, 128), f32),
            pltpu.VMEM((NCH, EC), jnp.int32),
            pltpu.VMEM((NCH, EC), jnp.int32),
            pltpu.VMEM((EC,), jnp.int32),
            pltpu.VMEM((EC,), jnp.int32),
            pltpu.VMEM_SHARED((N, SL, 128), f32),
            pltpu.VMEM_SHARED((N, SL, 128), f32),
            pltpu.VMEM_SHARED((N, SL, 128), f32),
            pltpu.SemaphoreType.DMA,
            pltpu.SemaphoreType.DMA,
        ],
    )
    return k(av4, bv4, src3, dst3, zrows)


def _tc_pre_body(x_ref, Wa_ref, ba_ref, av_o, bv_o):
    f32 = jnp.float32
    Wa = Wa_ref[...]
    xf = x_ref[...]
    av_o[...] = jnp.dot(xf, Wa[:2] - Wa[2:], preferred_element_type=f32,
                        precision=lax.Precision.HIGHEST) + ba_ref[...]
    bv_o[...] = jnp.dot(xf, Wa[2:], preferred_element_type=f32,
                        precision=lax.Precision.HIGHEST)


def _bn_from_partials(p_ref, ei_ref, Wb_ref, bb_ref, g_ref, be_ref):
    f32 = jnp.float32
    o = _dot3(p_ref[...], Wb_ref[...])
    dst_row = ei_ref[1:2, :]
    niota_c = lax.broadcasted_iota(jnp.int32, (N, E), 0)
    deg = jnp.sum((niota_c == dst_row).astype(f32), axis=1, keepdims=True)
    o = o.reshape(N, B, 128) + deg.reshape(N, 1, 1) * bb_ref[...]
    inv = 1.0 / (B * 128)
    mu = jnp.sum(jnp.sum(o, axis=1, keepdims=True), axis=2, keepdims=True) * inv
    d = o - mu
    var = jnp.sum(jnp.sum(d * d, axis=1, keepdims=True), axis=2, keepdims=True) * inv
    hn = d * lax.rsqrt(var + 1e-5) * g_ref[...] + be_ref[...]
    return jnp.maximum(hn, 0.0)


def _tc_mid_body(p_ref, ei_ref, Wb_ref, bb_ref, g_ref, be_ref,
                 Wa_ref, ba_ref, av_o, bv_o):
    h = _bn_from_partials(p_ref, ei_ref, Wb_ref, bb_ref, g_ref, be_ref)
    hf = h.reshape(N * B, 128)
    Wa = Wa_ref[...]
    av_o[...] = _dot3(hf, Wa[:128] - Wa[128:]) + ba_ref[...]
    bv_o[...] = _dot3(hf, Wa[128:])


def _tc_fin_body(p_ref, ei_ref, Wb_ref, bb_ref, g_ref, be_ref, h_o):
    h_o[...] = _bn_from_partials(p_ref, ei_ref, Wb_ref, bb_ref, g_ref, be_ref)


def _head_call(hf, Wm1, bm1, Wm2, bm2, Wm3, bm3, Wv, bv, Wadv, badv):
    f32 = jnp.float32
    Wadv_r = jnp.transpose(Wadv, (1, 0, 2)).reshape(256, 64)
    KC = 8
    KW = (N * N) // KC

    def cst(shape):
        return pl.BlockSpec(shape, lambda k: tuple(0 for _ in shape))

    return pl.pallas_call(
        _head_body,
        grid=(KC,),
        in_specs=[
            pl.BlockSpec((B, KW), lambda k: (0, k)),
            pl.BlockSpec((KW, 256), lambda k: (k, 0)),
            cst((1, 256)), cst((256, 256)), cst((1, 256)),
            cst((256, 256)), cst((1, 256)),
            cst((256, 1)), cst((1, 1)), cst((256, 64)), cst((1, 64)),
        ],
        out_specs=cst((B, 64)),
        out_shape=jax.ShapeDtypeStruct((B, 64), f32),
        scratch_shapes=[pltpu.VMEM((B, 256), f32)],
        compiler_params=pltpu.CompilerParams(
            dimension_semantics=("arbitrary",)),
    )(hf, Wm1, bm1.reshape(1, 256), Wm2, bm2.reshape(1, 256),
      Wm3, bm3.reshape(1, 256), Wv, bv.reshape(1, 1), Wadv_r, badv.reshape(1, 64))


def kernel(x, edge_index, W1a, b1a, W1b, b1b, W2a, b2a, W2b, b2b,
           W3a, b3a, W3b, b3b, g1, be1, g2, be2, g3, be3,
           Wm1, bm1, Wm2, bm2, Wm3, bm3, Wv, bv, Wadv, badv):
    f32 = jnp.float32
    x_nm = jnp.transpose(x, (1, 0, 2)).reshape(N * B, 2)
    eit = jnp.transpose(edge_index)
    src3 = edge_index[0].reshape(16, NCH, EC)
    dst3 = edge_index[1].reshape(16, NCH, EC)
    zrows = jnp.zeros((8, SL, 128), f32)

    def split_tables(t):
        t3 = t.reshape(N, B, 64)
        sc = jnp.transpose(t3[:, :S_SC, :].reshape(N, 2, SL, 128),
                           (1, 0, 2, 3))
        tc = jnp.transpose(t3[:, S_SC:, :], (1, 0, 2))
        return sc, tc

    def combine(p_sc, p_tc):
        a = jnp.transpose(p_sc, (1, 0, 2, 3)).reshape(N, S_SC, 64)
        b = p_tc.reshape(N, B_TC, 64)
        return jnp.concatenate([a, b], axis=1).reshape(N * B, 64)

    av, bvv = pl.pallas_call(
        _tc_pre_body,
        out_shape=(jax.ShapeDtypeStruct((N * B, 64), f32),
                   jax.ShapeDtypeStruct((N * B, 64), f32)),
    )(x_nm, W1a, b1a.reshape(1, 64))

    def mid(p, Wb, bb, g, be, Wa, ba):
        return pl.pallas_call(
            _tc_mid_body,
            out_shape=(jax.ShapeDtypeStruct((N * B, 64), f32),
                       jax.ShapeDtypeStruct((N * B, 64), f32)),
        )(p, edge_index, Wb, bb.reshape(1, 1, 128),
          g.reshape(N, 1, 1), be.reshape(N, 1, 1), Wa, ba.reshape(1, 64))

    def edge(av_t, bv_t):
        a_sc, a_tc = split_tables(av_t)
        b_sc, b_tc = split_tables(bv_t)
        p_sc = _sc_edge(a_sc, b_sc, src3, dst3, zrows)
        p_tc = _tc_edge(eit, edge_index, a_tc, b_tc)
        return combine(p_sc, p_tc)

    p = edge(av, bvv)
    av, bvv = mid(p, W1b, b1b, g1, be1, W2a, b2a)
    p = edge(av, bvv)
    av, bvv = mid(p, W2b, b2b, g2, be2, W3a, b3a)
    p = edge(av, bvv)
    h3 = pl.pallas_call(
        _tc_fin_body,
        out_shape=jax.ShapeDtypeStruct((N, B, 128), f32),
    )(p, edge_index, W3b, b3b.reshape(1, 1, 128),
      g3.reshape(N, 1, 1), be3.reshape(N, 1, 1))

    hf = jnp.transpose(h3, (1, 0, 2)).reshape(B, N * N)
    q = _head_call(hf, Wm1, bm1, Wm2, bm2, Wm3, bm3, Wv, bv, Wadv, badv)
    return q.reshape(B, 32, 2)


# speedup vs baseline: 1.4254x; 1.0032x over previous
import jax
import jax.numpy as jnp
from jax import lax
from jax.experimental import pallas as pl
from jax.experimental.pallas import tpu as pltpu
from jax.experimental.pallas import tpu_sc as plsc

B = 64
N = 128
E = 2048
G = 8


def _split(x):
    hi = x.astype(jnp.bfloat16)
    lo = (x - hi.astype(jnp.float32)).astype(jnp.bfloat16)
    return hi, lo


def _dot_oh3(oh_bf16, x):
    h1 = x.astype(jnp.bfloat16)
    r1 = x - h1.astype(jnp.float32)
    h2 = r1.astype(jnp.bfloat16)
    h3 = (r1 - h2.astype(jnp.float32)).astype(jnp.bfloat16)
    return (jnp.dot(oh_bf16, h1, preferred_element_type=jnp.float32)
            + jnp.dot(oh_bf16, h2, preferred_element_type=jnp.float32)
            + jnp.dot(oh_bf16, h3, preferred_element_type=jnp.float32))


def _dot3(x, w):
    xh, xl = _split(x)
    wh, wl = _split(w)
    return (jnp.dot(xh, wh, preferred_element_type=jnp.float32)
            + jnp.dot(xh, wl, preferred_element_type=jnp.float32)
            + jnp.dot(xl, wh, preferred_element_type=jnp.float32))


S_SC = 32
B_TC = B - S_SC


def _tc_edge_body(eit_ref, ei_ref, av_ref, bv_ref, out_ref):
    bf16 = jnp.bfloat16
    src_col = eit_ref[:, 0:1]
    dst_col = eit_ref[:, 1:2]
    dst_row = ei_ref[1:2, :]
    niota_r = lax.broadcasted_iota(jnp.int32, (E, N), 1)
    DS = jnp.concatenate([(dst_col == niota_r).astype(bf16),
                          (src_col == niota_r).astype(bf16)], axis=1)
    niota_c = lax.broadcasted_iota(jnp.int32, (N, E), 0)
    DT = (niota_c == dst_row).astype(bf16)
    for g in range(B_TC // G):
        a = jnp.concatenate(
            [av_ref[g * G + j] for j in range(G)], axis=1)
        bm = jnp.concatenate(
            [bv_ref[g * G + j] for j in range(G)], axis=1)
        pre = _dot_oh3(DS, jnp.concatenate([a, bm], axis=0))
        r = jnp.maximum(pre, 0.0)
        agg = _dot_oh3(DT, r)
        out_ref[:, pl.ds(g * G * 64, G * 64)] = agg


def _tc_edge(eit, ei, av_tc, bv_tc):
    return pl.pallas_call(
        _tc_edge_body,
        out_shape=jax.ShapeDtypeStruct((N, B_TC * 64), jnp.float32),
    )(eit, ei, av_tc, bv_tc)


def _head_body(hf_ref, Wm1_ref, bm1_ref, Wm2_ref, bm2_ref, Wm3_ref, bm3_ref,
               Wv_ref, bv_ref, Wadv_ref, badv_ref, out_ref, acc_ref):
    f32 = jnp.float32
    k = pl.program_id(0)

    @pl.when(k == 0)
    def _():
        acc_ref[...] = jnp.zeros_like(acc_ref)

    acc_ref[...] += _dot3(hf_ref[...], Wm1_ref[...])

    @pl.when(k == pl.num_programs(0) - 1)
    def _():
        z = jnp.maximum(acc_ref[...] + bm1_ref[...], 0.0)
        z = jnp.maximum(jnp.dot(z, Wm2_ref[...], preferred_element_type=f32, precision=lax.Precision.HIGHEST) + bm2_ref[...], 0.0)
        z = jnp.maximum(jnp.dot(z, Wm3_ref[...], preferred_element_type=f32, precision=lax.Precision.HIGHEST) + bm3_ref[...], 0.0)
        value = jnp.dot(z, Wv_ref[...], preferred_element_type=f32, precision=lax.Precision.HIGHEST) + bv_ref[...]
        adv = jnp.dot(z, Wadv_ref[...], preferred_element_type=f32, precision=lax.Precision.HIGHEST) + badv_ref[...]
        ii = lax.broadcasted_iota(jnp.int32, (64, 64), 0) // 2
        jj = lax.broadcasted_iota(jnp.int32, (64, 64), 1) // 2
        P = 0.5 * (ii == jj).astype(f32)
        out_ref[...] = value + adv - jnp.dot(adv, P, preferred_element_type=f32, precision=lax.Precision.HIGHEST)


SL = S_SC // 4
EC = 16
NCH = (E // 16) // EC


def _sc_edge_body(av_hbm, bv_hbm, src_hbm, dst_hbm, z_hbm, out_hbm,
                  ga, gb, didx, sidx, dj, sj, At, Bt, acc, sem_a, sem_b):
    cid = lax.axis_index("c")
    sid = lax.axis_index("s")
    rows = pl.ds(sid * 8, 8)
    pltpu.sync_copy(av_hbm.at[cid, rows], At.at[rows])
    pltpu.sync_copy(bv_hbm.at[cid, rows], Bt.at[rows])
    pltpu.sync_copy(z_hbm, acc.at[rows])
    pltpu.sync_copy(dst_hbm.at[sid], didx)
    pltpu.sync_copy(src_hbm.at[sid], sidx)
    plsc.subcore_barrier()
    for j in range(NCH):
        dj[...] = didx[j]
        sj[...] = sidx[j]
        ca = pltpu.async_copy(At.at[dj], ga, sem_a)
        cb = pltpu.async_copy(Bt.at[sj], gb, sem_b)
        ca.wait()
        cb.wait()

        def body(s, carry):
            for r in range(EC):
                for l in range(8):
                    ga[r, s, pl.ds(l * 16, 16)] = jnp.maximum(
                        ga[r, s, pl.ds(l * 16, 16)]
                        + gb[r, s, pl.ds(l * 16, 16)], 0.0)
            return carry

        lax.fori_loop(0, SL, body, 0)
        pltpu.sync_copy(ga, acc.at[dj], add=True)
    plsc.subcore_barrier()
    pltpu.sync_copy(acc.at[rows], out_hbm.at[cid, rows])


def _sc_edge(av4, bv4, src3, dst3, zrows):
    f32 = jnp.float32
    mesh = plsc.VectorSubcoreMesh(core_axis_name="c", subcore_axis_name="s")
    k = pl.kernel(
        _sc_edge_body,
        out_type=jax.ShapeDtypeStruct((2, N, SL, 128), f32),
        mesh=mesh,
        scratch_types=[
            pltpu.VMEM((EC, SL, 128), f32),
            pltpu.VMEM((EC, SL, 128), f32),
            pltpu.VMEM((NCH, EC), jnp.int32),
            pltpu.VMEM((NCH, EC), jnp.int32),
            pltpu.VMEM((EC,), jnp.int32),
            pltpu.VMEM((EC,), jnp.int32),
            pltpu.VMEM_SHARED((N, SL, 128), f32),
            pltpu.VMEM_SHARED((N, SL, 128), f32),
            pltpu.VMEM_SHARED((N, SL, 128), f32),
            pltpu.SemaphoreType.DMA,
            pltpu.SemaphoreType.DMA,
        ],
    )
    return k(av4, bv4, src3, dst3, zrows)


def _tc_pre_body(x_ref, Wa_ref, ba_ref, av_o, bv_o):
    f32 = jnp.float32
    Wa = Wa_ref[...]
    xf = x_ref[...]
    av_o[...] = jnp.dot(xf, Wa[:2] - Wa[2:], preferred_element_type=f32,
                        precision=lax.Precision.HIGHEST) + ba_ref[...]
    bv_o[...] = jnp.dot(xf, Wa[2:], preferred_element_type=f32,
                        precision=lax.Precision.HIGHEST)


def _bn_from_partials(p_ref, ei_ref, Wb_ref, bb_ref, g_ref, be_ref):
    f32 = jnp.float32
    o = _dot3(p_ref[...], Wb_ref[...])
    dst_row = ei_ref[1:2, :]
    niota_c = lax.broadcasted_iota(jnp.int32, (N, E), 0)
    deg = jnp.sum((niota_c == dst_row).astype(f32), axis=1, keepdims=True)
    o = o.reshape(N, B, 128) + deg.reshape(N, 1, 1) * bb_ref[...]
    inv = 1.0 / (B * 128)
    mu = jnp.sum(jnp.sum(o, axis=1, keepdims=True), axis=2, keepdims=True) * inv
    d = o - mu
    var = jnp.sum(jnp.sum(d * d, axis=1, keepdims=True), axis=2, keepdims=True) * inv
    hn = d * lax.rsqrt(var + 1e-5) * g_ref[...] + be_ref[...]
    return jnp.maximum(hn, 0.0)


def _tc_mid_body(p_ref, ei_ref, Wb_ref, bb_ref, g_ref, be_ref,
                 Wa_ref, ba_ref, av_o, bv_o):
    h = _bn_from_partials(p_ref, ei_ref, Wb_ref, bb_ref, g_ref, be_ref)
    hf = h.reshape(N * B, 128)
    Wa = Wa_ref[...]
    av_o[...] = _dot3(hf, Wa[:128] - Wa[128:]) + ba_ref[...]
    bv_o[...] = _dot3(hf, Wa[128:])


def _tc_fin_body(p_ref, ei_ref, Wb_ref, bb_ref, g_ref, be_ref, h_o):
    h_o[...] = _bn_from_partials(p_ref, ei_ref, Wb_ref, bb_ref, g_ref, be_ref)


def _head_call(hf, Wm1, bm1, Wm2, bm2, Wm3, bm3, Wv, bv, Wadv, badv):
    f32 = jnp.float32
    Wadv_r = jnp.transpose(Wadv, (1, 0, 2)).reshape(256, 64)
    KC = 8
    KW = (N * N) // KC

    def cst(shape):
        return pl.BlockSpec(shape, lambda k: tuple(0 for _ in shape))

    return pl.pallas_call(
        _head_body,
        grid=(KC,),
        in_specs=[
            pl.BlockSpec((B, KW), lambda k: (0, k)),
            pl.BlockSpec((KW, 256), lambda k: (k, 0)),
            cst((1, 256)), cst((256, 256)), cst((1, 256)),
            cst((256, 256)), cst((1, 256)),
            cst((256, 1)), cst((1, 1)), cst((256, 64)), cst((1, 64)),
        ],
        out_specs=cst((B, 64)),
        out_shape=jax.ShapeDtypeStruct((B, 64), f32),
        scratch_shapes=[pltpu.VMEM((B, 256), f32)],
        compiler_params=pltpu.CompilerParams(
            dimension_semantics=("arbitrary",)),
    )(hf, Wm1, bm1.reshape(1, 256), Wm2, bm2.reshape(1, 256),
      Wm3, bm3.reshape(1, 256), Wv, bv.reshape(1, 1), Wadv_r, badv.reshape(1, 64))


def kernel(x, edge_index, W1a, b1a, W1b, b1b, W2a, b2a, W2b, b2b,
           W3a, b3a, W3b, b3b, g1, be1, g2, be2, g3, be3,
           Wm1, bm1, Wm2, bm2, Wm3, bm3, Wv, bv, Wadv, badv):
    f32 = jnp.float32
    x_nm = jnp.transpose(x, (1, 0, 2)).reshape(N * B, 2)
    eit = jnp.transpose(edge_index)
    src3 = edge_index[0].reshape(16, NCH, EC)
    dst3 = edge_index[1].reshape(16, NCH, EC)
    zrows = jnp.zeros((8, SL, 128), f32)

    def split_tables(t):
        t3 = t.reshape(N, B, 64)
        sc = jnp.transpose(t3[:, :S_SC, :].reshape(N, 2, SL, 128),
                           (1, 0, 2, 3))
        tc = jnp.transpose(t3[:, S_SC:, :], (1, 0, 2))
        return sc, tc

    def combine(p_sc, p_tc):
        a = jnp.transpose(p_sc, (1, 0, 2, 3)).reshape(N, S_SC, 64)
        b = p_tc.reshape(N, B_TC, 64)
        return jnp.concatenate([a, b], axis=1).reshape(N * B, 64)

    av, bvv = pl.pallas_call(
        _tc_pre_body,
        out_shape=(jax.ShapeDtypeStruct((N * B, 64), f32),
                   jax.ShapeDtypeStruct((N * B, 64), f32)),
    )(x_nm, W1a, b1a.reshape(1, 64))

    def mid(p, Wb, bb, g, be, Wa, ba):
        return pl.pallas_call(
            _tc_mid_body,
            out_shape=(jax.ShapeDtypeStruct((N * B, 64), f32),
                       jax.ShapeDtypeStruct((N * B, 64), f32)),
        )(p, edge_index, Wb, bb.reshape(1, 1, 128),
          g.reshape(N, 1, 1), be.reshape(N, 1, 1), Wa, ba.reshape(1, 64))

    def edge(av_t, bv_t):
        a_sc, a_tc = split_tables(av_t)
        b_sc, b_tc = split_tables(bv_t)
        p_sc = _sc_edge(a_sc, b_sc, src3, dst3, zrows)
        p_tc = _tc_edge(eit, edge_index, a_tc, b_tc)
        return combine(p_sc, p_tc)

    p = edge(av, bvv)
    av, bvv = mid(p, W1b, b1b, g1, be1, W2a, b2a)
    p = edge(av, bvv)
    av, bvv = mid(p, W2b, b2b, g2, be2, W3a, b3a)
    p = edge(av, bvv)
    h3 = pl.pallas_call(
        _tc_fin_body,
        out_shape=jax.ShapeDtypeStruct((N, B, 128), f32),
    )(p, edge_index, W3b, b3b.reshape(1, 1, 128),
      g3.reshape(N, 1, 1), be3.reshape(N, 1, 1))

    hf = jnp.transpose(h3, (1, 0, 2)).reshape(B, N * N)
    q = _head_call(hf, Wm1, bm1, Wm2, bm2, Wm3, bm3, Wv, bv, Wadv, badv)
    return q.reshape(B, 32, 2)
